# baseline (device time: 136413 ns/iter reference)
import jax
import jax.numpy as jnp
from jax import lax
from jax.experimental import pallas as pl
from jax.experimental.pallas import tpu as pltpu

N_DEV = 4
N_TOK = 2048
D = 1024
H = 1024
N_EXP = 32
EXP_PER_DEV = N_EXP // N_DEV
CAP = 51
CAP_PAD = 64
ROWS = EXP_PER_DEV * CAP_PAD
CHUNK = N_TOK // N_DEV
TRASH = N_TOK


def _fused(x, expert_W, tok, dst):

    def body(x_ref, ew_ref, tok_ref, dst_ref, out_ref,
             xg_ref, y_ref, part_ref, comm_ref, send_sems, recv_sems):
        d = lax.axis_index("i")
        left = lax.rem(d + N_DEV - 1, N_DEV)
        right = lax.rem(d + 1, N_DEV)

        barrier = pltpu.get_barrier_semaphore()
        for nbr in (left, right):
            pl.semaphore_signal(
                barrier, inc=1, device_id=(nbr,),
                device_id_type=pl.DeviceIdType.MESH,
            )
        pl.semaphore_wait(barrier, 2)

        def gath(s, _):
            t = tok_ref[s]
            xg_ref[pl.ds(s, 1), :] = x_ref[pl.ds(t, 1), :]
            return 0
        lax.fori_loop(0, ROWS, gath, 0, unroll=8)

        for e in range(EXP_PER_DEV):
            y_ref[pl.ds(e * CAP_PAD, CAP_PAD), :] = jnp.dot(
                xg_ref[pl.ds(e * CAP_PAD, CAP_PAD), :], ew_ref[e],
                preferred_element_type=jnp.float32,
            )

        part_ref[...] = jnp.zeros_like(part_ref)
        def scat(s, _):
            t = dst_ref[s]
            part_ref[pl.ds(t, 1), :] = y_ref[pl.ds(s, 1), :]
            return 0
        lax.fori_loop(0, ROWS, scat, 0, unroll=8)

        c0 = lax.rem(d + N_DEV - 1, N_DEV)
        comm_ref[0] = part_ref[pl.ds(c0 * CHUNK, CHUNK), :]
        for s in range(N_DEV - 1):
            rdma = pltpu.make_async_remote_copy(
                src_ref=comm_ref.at[s],
                dst_ref=comm_ref.at[s + 1],
                send_sem=send_sems.at[s],
                recv_sem=recv_sems.at[s],
                device_id=(right,),
                device_id_type=pl.DeviceIdType.MESH,
            )
            rdma.start()
            rdma.wait()
            c = lax.rem(d + 2 * N_DEV - s - 2, N_DEV)
            comm_ref[s + 1] = comm_ref[s + 1] + part_ref[pl.ds(c * CHUNK, CHUNK), :]
        out_ref[...] = comm_ref[N_DEV - 1]

    return pl.pallas_call(
        body,
        out_shape=jax.ShapeDtypeStruct((CHUNK, H), jnp.float32),
        in_specs=[
            pl.BlockSpec(memory_space=pltpu.VMEM),
            pl.BlockSpec(memory_space=pltpu.VMEM),
            pl.BlockSpec(memory_space=pltpu.SMEM),
            pl.BlockSpec(memory_space=pltpu.SMEM),
        ],
        out_specs=pl.BlockSpec(memory_space=pltpu.VMEM),
        scratch_shapes=[
            pltpu.VMEM((ROWS, D), jnp.float32),
            pltpu.VMEM((ROWS, H), jnp.float32),
            pltpu.VMEM((N_TOK + 8, H), jnp.float32),
            pltpu.VMEM((N_DEV, CHUNK, H), jnp.float32),
            pltpu.SemaphoreType.DMA((N_DEV - 1,)),
            pltpu.SemaphoreType.DMA((N_DEV - 1,)),
        ],
        compiler_params=pltpu.CompilerParams(
            collective_id=0,
            vmem_limit_bytes=100 * 1024 * 1024,
        ),
    )(x, expert_W, tok, dst)


def kernel(x, router_W, route_idx, expert_W):
    del router_W
    pos = lax.axis_index("i")

    e = route_idx[:, 0].astype(jnp.int32)
    oh = (e[:, None] == jnp.arange(N_EXP, dtype=jnp.int32)[None, :]).astype(jnp.int32)
    rank = jnp.take_along_axis(jnp.cumsum(oh, axis=0), e[:, None], axis=1)[:, 0] - 1

    local_e = e - EXP_PER_DEV * pos
    mine = (local_e >= 0) & (local_e < EXP_PER_DEV) & (rank < CAP)
    tokens = jnp.arange(N_TOK, dtype=jnp.int32)
    flat = jnp.where(mine, local_e * CAP_PAD + rank, ROWS)
    tok = jnp.zeros((ROWS + 1,), jnp.int32).at[flat].set(tokens)[:ROWS]
    valid = jnp.zeros((ROWS + 1,), jnp.bool_).at[flat].set(True)[:ROWS]
    dst = jnp.where(valid, tok, TRASH)

    return _fused(x, expert_W, tok, dst)


# device time: 108572 ns/iter; 1.2564x vs baseline; 1.2564x over previous
import jax
import jax.numpy as jnp
from jax import lax
from jax.experimental import pallas as pl
from jax.experimental.pallas import tpu as pltpu

N_DEV = 4
N_TOK = 2048
D = 1024
H = 1024
N_EXP = 32
EXP_PER_DEV = N_EXP // N_DEV
CAP = 51
CAP_PAD = 64
ROWS = EXP_PER_DEV * CAP_PAD
CHUNK = N_TOK // N_DEV
TRASH = N_TOK


def _fused(x, expert_W, tok, dst):

    def body(x_ref, ew_ref, tok_ref, dst_ref, out_ref,
             xg_ref, y_ref, part_ref, comm_ref, send_sems, recv_sems):
        d = lax.axis_index("i")
        left = lax.rem(d + N_DEV - 1, N_DEV)
        right = lax.rem(d + 1, N_DEV)

        barrier = pltpu.get_barrier_semaphore()
        for nbr in (left, right):
            pl.semaphore_signal(
                barrier, inc=1, device_id=(nbr,),
                device_id_type=pl.DeviceIdType.MESH,
            )
        pl.semaphore_wait(barrier, 2)

        def gath(s, _):
            t = tok_ref[s]
            xg_ref[pl.ds(s, 1), :] = x_ref[pl.ds(t, 1), :]
            return 0
        lax.fori_loop(0, ROWS, gath, 0, unroll=8)

        for e in range(EXP_PER_DEV):
            y_ref[pl.ds(e * CAP_PAD, CAP_PAD), :] = jnp.dot(
                xg_ref[pl.ds(e * CAP_PAD, CAP_PAD), :], ew_ref[e],
                preferred_element_type=jnp.float32,
            )

        part_ref[...] = jnp.zeros_like(part_ref)
        def scat(s, _):
            t = dst_ref[s]
            part_ref[pl.ds(t, 1), :] = y_ref[pl.ds(s, 1), :]
            return 0
        lax.fori_loop(0, ROWS, scat, 0, unroll=8)

        c0 = lax.rem(d + N_DEV - 1, N_DEV)
        comm_ref[0] = part_ref[pl.ds(c0 * CHUNK, CHUNK), :]
        for s in range(N_DEV - 1):
            rdma = pltpu.make_async_remote_copy(
                src_ref=comm_ref.at[s],
                dst_ref=comm_ref.at[s + 1],
                send_sem=send_sems.at[s],
                recv_sem=recv_sems.at[s],
                device_id=(right,),
                device_id_type=pl.DeviceIdType.MESH,
            )
            rdma.start()
            rdma.wait()
            c = lax.rem(d + 2 * N_DEV - s - 2, N_DEV)
            comm_ref[s + 1] = comm_ref[s + 1] + part_ref[pl.ds(c * CHUNK, CHUNK), :]
        out_ref[...] = comm_ref[N_DEV - 1]

    return pl.pallas_call(
        body,
        out_shape=jax.ShapeDtypeStruct((CHUNK, H), jnp.float32),
        in_specs=[
            pl.BlockSpec(memory_space=pltpu.VMEM),
            pl.BlockSpec(memory_space=pltpu.VMEM),
            pl.BlockSpec(memory_space=pltpu.SMEM),
            pl.BlockSpec(memory_space=pltpu.SMEM),
        ],
        out_specs=pl.BlockSpec(memory_space=pltpu.VMEM),
        scratch_shapes=[
            pltpu.VMEM((ROWS, D), jnp.float32),
            pltpu.VMEM((ROWS, H), jnp.float32),
            pltpu.VMEM((N_TOK + 8, H), jnp.float32),
            pltpu.VMEM((N_DEV, CHUNK, H), jnp.float32),
            pltpu.SemaphoreType.DMA((N_DEV - 1,)),
            pltpu.SemaphoreType.DMA((N_DEV - 1,)),
        ],
        compiler_params=pltpu.CompilerParams(
            collective_id=0,
            vmem_limit_bytes=100 * 1024 * 1024,
        ),
    )(x, expert_W, tok, dst)


def kernel(x, router_W, route_idx, expert_W):
    del router_W
    pos = lax.axis_index("i")

    e = route_idx[:, 0].astype(jnp.int32)
    tokens = jnp.arange(N_TOK, dtype=jnp.int32)
    same = (e[:, None] == e[None, :]) & (tokens[:, None] > tokens[None, :])
    rank = jnp.sum(same, axis=1, dtype=jnp.int32)

    local_e = e - EXP_PER_DEV * pos
    mine = (local_e >= 0) & (local_e < EXP_PER_DEV) & (rank < CAP)
    flat = jnp.where(mine, local_e * CAP_PAD + rank, ROWS)
    eq = flat[None, :] == jnp.arange(ROWS, dtype=jnp.int32)[:, None]
    tok = jnp.sum(eq * tokens[None, :], axis=1, dtype=jnp.int32)
    valid = jnp.any(eq, axis=1)
    dst = jnp.where(valid, tok, TRASH)

    return _fused(x, expert_W, tok, dst)


# device time: 75026 ns/iter; 1.8182x vs baseline; 1.4471x over previous
import jax
import jax.numpy as jnp
from jax import lax
from jax.experimental import pallas as pl
from jax.experimental.pallas import tpu as pltpu

N_DEV = 4
N_TOK = 2048
D = 1024
H = 1024
N_EXP = 32
EXP_PER_DEV = N_EXP // N_DEV
CAP = 51
CAP_PAD = 64
ROWS = EXP_PER_DEV * CAP_PAD
CHUNK = N_TOK // N_DEV
HALF = CHUNK // 2
TRASH = N_TOK


def _fused(x, expert_W, tok, dst):

    def body(x_ref, ew_ref, tok_ref, dst_ref, out_ref,
             xg_ref, y_ref, part_ref, commr_ref, comml_ref,
             sendr_sems, recvr_sems, sendl_sems, recvl_sems):
        d = lax.axis_index("i")
        left = lax.rem(d + N_DEV - 1, N_DEV)
        right = lax.rem(d + 1, N_DEV)

        barrier = pltpu.get_barrier_semaphore()
        for nbr in (left, right):
            pl.semaphore_signal(
                barrier, inc=1, device_id=(nbr,),
                device_id_type=pl.DeviceIdType.MESH,
            )
        pl.semaphore_wait(barrier, 2)

        def gath(s, _):
            t = tok_ref[s]
            xg_ref[pl.ds(s, 1), :] = x_ref[pl.ds(t, 1), :]
            return 0
        lax.fori_loop(0, ROWS, gath, 0, unroll=8)

        for e in range(EXP_PER_DEV):
            y_ref[pl.ds(e * CAP_PAD, CAP_PAD), :] = jnp.dot(
                xg_ref[pl.ds(e * CAP_PAD, CAP_PAD), :], ew_ref[e],
                preferred_element_type=jnp.float32,
            )

        part_ref[...] = jnp.zeros_like(part_ref)
        def scat(s, _):
            t = dst_ref[s]
            part_ref[pl.ds(t, 1), :] = y_ref[pl.ds(s, 1), :]
            return 0
        lax.fori_loop(0, ROWS, scat, 0, unroll=8)

        cr = lax.rem(d + N_DEV - 1, N_DEV)
        cl = lax.rem(d + 1, N_DEV)
        commr_ref[0] = part_ref[pl.ds(cr * CHUNK, HALF), :]
        comml_ref[0] = part_ref[pl.ds(cl * CHUNK + HALF, HALF), :]
        for s in range(N_DEV - 1):
            rdma_r = pltpu.make_async_remote_copy(
                src_ref=commr_ref.at[s],
                dst_ref=commr_ref.at[s + 1],
                send_sem=sendr_sems.at[s],
                recv_sem=recvr_sems.at[s],
                device_id=(right,),
                device_id_type=pl.DeviceIdType.MESH,
            )
            rdma_l = pltpu.make_async_remote_copy(
                src_ref=comml_ref.at[s],
                dst_ref=comml_ref.at[s + 1],
                send_sem=sendl_sems.at[s],
                recv_sem=recvl_sems.at[s],
                device_id=(left,),
                device_id_type=pl.DeviceIdType.MESH,
            )
            rdma_r.start()
            rdma_l.start()
            rdma_r.wait()
            rdma_l.wait()
            cr = lax.rem(d + 2 * N_DEV - s - 2, N_DEV)
            cl = lax.rem(d + s + 2, N_DEV)
            commr_ref[s + 1] = (
                commr_ref[s + 1] + part_ref[pl.ds(cr * CHUNK, HALF), :]
            )
            comml_ref[s + 1] = (
                comml_ref[s + 1] + part_ref[pl.ds(cl * CHUNK + HALF, HALF), :]
            )
        out_ref[pl.ds(0, HALF), :] = commr_ref[N_DEV - 1]
        out_ref[pl.ds(HALF, HALF), :] = comml_ref[N_DEV - 1]

    return pl.pallas_call(
        body,
        out_shape=jax.ShapeDtypeStruct((CHUNK, H), jnp.float32),
        in_specs=[
            pl.BlockSpec(memory_space=pltpu.VMEM),
            pl.BlockSpec(memory_space=pltpu.VMEM),
            pl.BlockSpec(memory_space=pltpu.SMEM),
            pl.BlockSpec(memory_space=pltpu.SMEM),
        ],
        out_specs=pl.BlockSpec(memory_space=pltpu.VMEM),
        scratch_shapes=[
            pltpu.VMEM((ROWS, D), jnp.float32),
            pltpu.VMEM((ROWS, H), jnp.float32),
            pltpu.VMEM((N_TOK + 8, H), jnp.float32),
            pltpu.VMEM((N_DEV, HALF, H), jnp.float32),
            pltpu.VMEM((N_DEV, HALF, H), jnp.float32),
            pltpu.SemaphoreType.DMA((N_DEV - 1,)),
            pltpu.SemaphoreType.DMA((N_DEV - 1,)),
            pltpu.SemaphoreType.DMA((N_DEV - 1,)),
            pltpu.SemaphoreType.DMA((N_DEV - 1,)),
        ],
        compiler_params=pltpu.CompilerParams(
            collective_id=0,
            vmem_limit_bytes=100 * 1024 * 1024,
        ),
    )(x, expert_W, tok, dst)


def kernel(x, router_W, route_idx, expert_W):
    del router_W
    pos = lax.axis_index("i")

    e = route_idx[:, 0].astype(jnp.int32)
    tokens = jnp.arange(N_TOK, dtype=jnp.int32)
    same = (e[:, None] == e[None, :]) & (tokens[:, None] > tokens[None, :])
    rank = jnp.sum(same, axis=1, dtype=jnp.int32)

    local_e = e - EXP_PER_DEV * pos
    mine = (local_e >= 0) & (local_e < EXP_PER_DEV) & (rank < CAP)
    flat = jnp.where(mine, local_e * CAP_PAD + rank, ROWS)
    eq = flat[None, :] == jnp.arange(ROWS, dtype=jnp.int32)[:, None]
    tok = jnp.sum(eq * tokens[None, :], axis=1, dtype=jnp.int32)
    valid = jnp.any(eq, axis=1)
    dst = jnp.where(valid, tok, TRASH)

    return _fused(x, expert_W, tok, dst)


# device time: 47443 ns/iter; 2.8753x vs baseline; 1.5814x over previous
import jax
import jax.numpy as jnp
from jax import lax
from jax.experimental import pallas as pl
from jax.experimental.pallas import tpu as pltpu

N_DEV = 4
N_TOK = 2048
D = 1024
H = 1024
N_EXP = 32
EXP_PER_DEV = N_EXP // N_DEV
CAP = 51
CAP_PAD = 64
ROWS = EXP_PER_DEV * CAP_PAD
CHUNK = N_TOK // N_DEV
HALF = CHUNK // 2
TRASH = N_TOK
NB = 4
COLB = H // NB
NB_RS = 8
COLB_RS = H // NB_RS
S = N_DEV - 1


def _fused(x, expert_W, tok, dst):

    def body(x_ref, ew_ref, tok_ref, dst_ref, out_ref,
             xg_ref, ew_vmem, y_ref, part_ref, stgr_ref, stgl_ref,
             commr_ref, comml_ref,
             ew_sems, sendr_sems, recvr_sems, sendl_sems, recvl_sems):
        d = lax.axis_index("i")
        left = lax.rem(d + N_DEV - 1, N_DEV)
        right = lax.rem(d + 1, N_DEV)

        barrier = pltpu.get_barrier_semaphore()
        for nbr in (left, right):
            pl.semaphore_signal(
                barrier, inc=1, device_id=(nbr,),
                device_id_type=pl.DeviceIdType.MESH,
            )

        ew_cp = []
        for b in range(NB):
            cp = pltpu.make_async_copy(
                ew_ref.at[:, :, pl.ds(b * COLB, COLB)],
                ew_vmem.at[b],
                ew_sems.at[b],
            )
            cp.start()
            ew_cp.append(cp)

        def gath(s_, _):
            t = tok_ref[s_]
            xg_ref[pl.ds(s_, 1), :] = x_ref[pl.ds(t, 1), :]
            return 0
        lax.fori_loop(0, ROWS, gath, 0, unroll=8)

        part_ref[...] = jnp.zeros_like(part_ref)

        def compute_block(b):
            ew_cp[b].wait()
            for e in range(EXP_PER_DEV):
                y_ref[pl.ds(e * CAP_PAD, CAP_PAD), pl.ds(b * COLB, COLB)] = (
                    jnp.dot(
                        xg_ref[pl.ds(e * CAP_PAD, CAP_PAD), :], ew_vmem[b, e],
                        preferred_element_type=jnp.float32,
                    )
                )

            def scat(s_, _):
                t = dst_ref[s_]
                part_ref[pl.ds(t, 1), pl.ds(b * COLB, COLB)] = (
                    y_ref[pl.ds(s_, 1), pl.ds(b * COLB, COLB)]
                )
                return 0
            lax.fori_loop(0, ROWS, scat, 0, unroll=8)

        def make_rdmas(b, s):
            if s == 0:
                src_r = stgr_ref.at[b]
                src_l = stgl_ref.at[b]
            else:
                src_r = commr_ref.at[b, s - 1]
                src_l = comml_ref.at[b, s - 1]
            rdma_r = pltpu.make_async_remote_copy(
                src_ref=src_r,
                dst_ref=commr_ref.at[b, s],
                send_sem=sendr_sems.at[b, s],
                recv_sem=recvr_sems.at[b, s],
                device_id=(right,),
                device_id_type=pl.DeviceIdType.MESH,
            )
            rdma_l = pltpu.make_async_remote_copy(
                src_ref=src_l,
                dst_ref=comml_ref.at[b, s],
                send_sem=sendl_sems.at[b, s],
                recv_sem=recvl_sems.at[b, s],
                device_id=(left,),
                device_id_type=pl.DeviceIdType.MESH,
            )
            return rdma_r, rdma_l

        rdmas = {}

        def start_hop(b, s):
            if s == 0:
                cr0 = lax.rem(d + N_DEV - 1, N_DEV)
                cl0 = lax.rem(d + 1, N_DEV)
                stgr_ref[b] = part_ref[
                    pl.ds(cr0 * CHUNK, HALF), pl.ds(b * COLB_RS, COLB_RS)
                ].astype(jnp.bfloat16)
                stgl_ref[b] = part_ref[
                    pl.ds(cl0 * CHUNK + HALF, HALF),
                    pl.ds(b * COLB_RS, COLB_RS)
                ].astype(jnp.bfloat16)
            rdmas[(b, s)] = make_rdmas(b, s)
            rdmas[(b, s)][0].start()
            rdmas[(b, s)][1].start()

        def finish_hop(b, s):
            rdma_r, rdma_l = rdmas[(b, s)]
            rdma_r.wait()
            rdma_l.wait()
            cr = lax.rem(d + 2 * N_DEV - s - 2, N_DEV)
            cl = lax.rem(d + s + 2, N_DEV)
            acc_r = (
                commr_ref[b, s].astype(jnp.float32)
                + part_ref[pl.ds(cr * CHUNK, HALF),
                           pl.ds(b * COLB_RS, COLB_RS)]
            )
            acc_l = (
                comml_ref[b, s].astype(jnp.float32)
                + part_ref[pl.ds(cl * CHUNK + HALF, HALF),
                           pl.ds(b * COLB_RS, COLB_RS)]
            )
            if s + 1 < S:
                commr_ref[b, s] = acc_r.astype(jnp.bfloat16)
                comml_ref[b, s] = acc_l.astype(jnp.bfloat16)
                start_hop(b, s + 1)
            else:
                out_ref[pl.ds(0, HALF), pl.ds(b * COLB_RS, COLB_RS)] = acc_r
                out_ref[pl.ds(HALF, HALF), pl.ds(b * COLB_RS, COLB_RS)] = (
                    acc_l
                )

        start_step = {
            b: ((b + 1) * NB + NB_RS - 1) // NB_RS - 1 for b in range(NB_RS)
        }
        events = []
        for step in range(NB + S + NB_RS):
            if step < NB:
                events.append(("compute", step))
                for b in range(NB_RS):
                    if start_step[b] == step:
                        events.append(("start0", b))
            for b in range(NB_RS):
                s = step - start_step[b] - 1
                if 0 <= s < S:
                    events.append(("finish", b, s))

        for ev in events:
            if ev[0] == "compute":
                compute_block(ev[1])
            elif ev[0] == "start0":
                if ev[1] == 0:
                    pl.semaphore_wait(barrier, 2)
                start_hop(ev[1], 0)
            else:
                finish_hop(ev[1], ev[2])

    return pl.pallas_call(
        body,
        out_shape=jax.ShapeDtypeStruct((CHUNK, H), jnp.float32),
        in_specs=[
            pl.BlockSpec(memory_space=pltpu.VMEM),
            pl.BlockSpec(memory_space=pl.ANY),
            pl.BlockSpec(memory_space=pltpu.SMEM),
            pl.BlockSpec(memory_space=pltpu.SMEM),
        ],
        out_specs=pl.BlockSpec(memory_space=pltpu.VMEM),
        scratch_shapes=[
            pltpu.VMEM((ROWS, D), jnp.float32),
            pltpu.VMEM((NB, EXP_PER_DEV, D, COLB), jnp.float32),
            pltpu.VMEM((ROWS, H), jnp.float32),
            pltpu.VMEM((N_TOK + 8, H), jnp.float32),
            pltpu.VMEM((NB_RS, HALF, COLB_RS), jnp.bfloat16),
            pltpu.VMEM((NB_RS, HALF, COLB_RS), jnp.bfloat16),
            pltpu.VMEM((NB_RS, S, HALF, COLB_RS), jnp.bfloat16),
            pltpu.VMEM((NB_RS, S, HALF, COLB_RS), jnp.bfloat16),
            pltpu.SemaphoreType.DMA((NB,)),
            pltpu.SemaphoreType.DMA((NB_RS, S)),
            pltpu.SemaphoreType.DMA((NB_RS, S)),
            pltpu.SemaphoreType.DMA((NB_RS, S)),
            pltpu.SemaphoreType.DMA((NB_RS, S)),
        ],
        compiler_params=pltpu.CompilerParams(
            collective_id=0,
            vmem_limit_bytes=100 * 1024 * 1024,
        ),
    )(x, expert_W, tok, dst)


def kernel(x, router_W, route_idx, expert_W):
    del router_W
    pos = lax.axis_index("i")

    e = route_idx[:, 0].astype(jnp.int32)
    tokens = jnp.arange(N_TOK, dtype=jnp.int32)
    nbk, bk = 16, N_TOK // 16
    eb = e.reshape(nbk, bk)
    ohb = eb[:, :, None] == jnp.arange(N_EXP, dtype=jnp.int32)[None, None, :]
    blk_cnt = jnp.sum(ohb, axis=1, dtype=jnp.float32)
    tri = (jnp.arange(nbk)[:, None] > jnp.arange(nbk)[None, :]).astype(
        jnp.float32
    )
    blk_cum = tri @ blk_cnt
    prefix = jnp.sum(
        ohb * blk_cum[:, None, :], axis=2, dtype=jnp.float32
    ).astype(jnp.int32)
    idx = jnp.arange(bk, dtype=jnp.int32)
    within = jnp.sum(
        (eb[:, :, None] == eb[:, None, :])
        & (idx[:, None] > idx[None, :])[None],
        axis=2, dtype=jnp.int32,
    )
    rank = (prefix + within).reshape(N_TOK)

    local_e = e - EXP_PER_DEV * pos
    mine = (local_e >= 0) & (local_e < EXP_PER_DEV) & (rank < CAP)
    flat = jnp.where(mine, local_e * CAP_PAD + rank, ROWS)
    eq = flat[None, :] == jnp.arange(ROWS, dtype=jnp.int32)[:, None]
    tok = jnp.sum(eq * tokens[None, :], axis=1, dtype=jnp.int32)
    valid = jnp.any(eq, axis=1)
    dst = jnp.where(valid, tok, TRASH)

    return _fused(x, expert_W, tok, dst)
